# baseline (speedup 1.0000x reference)
import functools

import jax
import jax.numpy as jnp
from jax import lax
from jax.experimental import pallas as pl
from jax.experimental.pallas import tpu as pltpu
from jax.experimental.pallas import tpu_sc as plsc

_B, _Q, _C, _T, _K = 8, 900, 91, 300, 6
_COST_CLASS, _COST_BBOX, _COST_GIOU = 1.0, 5.0, 2.0
_EPS = 1e-06
_ALPHA = 0.25
_L = 16
_NCHUNK = 57
_QPAD = _NCHUNK * _L
_NW = 32
_ROWS = _B * _T
_BLK = 4
_NBLK = 19
_RPW = _BLK * _NBLK
_NG = 8
_GS = 8


def _cost_kernel(logits_ref, pbT_ref, labels_ref, tb_ref, cost_ref, idxt_ref):
    logits = jnp.nan_to_num(logits_ref[0], nan=0.0)
    prob = jax.nn.sigmoid(logits)
    pos = _ALPHA * ((1.0 - prob) * (1.0 - prob)) * -jnp.log(prob + 1e-08)
    neg = (1.0 - _ALPHA) * (prob * prob) * -jnp.log(1.0 - prob + 1e-08)
    d = pos - neg
    labels = labels_ref[0]
    onehot = (labels == jax.lax.broadcasted_iota(jnp.int32, (_T, _C), 1)
              ).astype(jnp.float32)
    c_cls = jax.lax.dot_general(
        onehot, d, (((1,), (1,)), ((), ())),
        preferred_element_type=jnp.float32,
        precision=jax.lax.Precision.HIGHEST)

    pbT = jax.nn.sigmoid(pbT_ref[0])
    qcx, qcy = pbT[0:1, :], pbT[1:2, :]
    qw, qh = pbT[2:3, :], pbT[3:4, :]
    tb = jnp.clip(tb_ref[0], 0.0, 1.0)
    tx1, ty1 = tb[:, 0:1], tb[:, 1:2]
    tx2, ty2 = tb[:, 2:3], tb[:, 3:4]
    tw = jnp.maximum(tx2 - tx1, 1e-05)
    th = jnp.maximum(ty2 - ty1, 1e-05)
    tcx = (tx1 + tx2) * 0.5
    tcy = (ty1 + ty2) * 0.5

    c_l1 = (jnp.abs(qcx - tcx) + jnp.abs(qcy - tcy)
            + jnp.abs(qw - tw) + jnp.abs(qh - th))

    qx1 = jnp.clip(qcx - 0.5 * qw, 0.0, 1.0)
    qy1 = jnp.clip(qcy - 0.5 * qh, 0.0, 1.0)
    qx2 = jnp.clip(qcx + 0.5 * qw, 0.0, 1.0)
    qy2 = jnp.clip(qcy + 0.5 * qh, 0.0, 1.0)
    lt_x = jnp.maximum(qx1, tx1)
    lt_y = jnp.maximum(qy1, ty1)
    rb_x = jnp.minimum(qx2, tx2)
    rb_y = jnp.minimum(qy2, ty2)
    inter = jnp.maximum(rb_x - lt_x, 0.0) * jnp.maximum(rb_y - lt_y, 0.0)
    area_q = jnp.maximum(qx2 - qx1, 0.0) * jnp.maximum(qy2 - qy1, 0.0)
    area_t = jnp.maximum(tx2 - tx1, 0.0) * jnp.maximum(ty2 - ty1, 0.0)
    union = jnp.maximum(area_q + area_t - inter, _EPS)
    iou = inter / union
    en_x = jnp.maximum(qx2, tx2) - jnp.minimum(qx1, tx1)
    en_y = jnp.maximum(qy2, ty2) - jnp.minimum(qy1, ty1)
    area_c = jnp.maximum(jnp.maximum(en_x, 0.0) * jnp.maximum(en_y, 0.0), _EPS)
    giou = jnp.clip(iou - (area_c - union) / area_c, -1.0, 1.0)
    c_iou = 1.0 - giou

    cost = _COST_CLASS * c_cls + _COST_BBOX * c_l1 + _COST_GIOU * c_iou
    cost_ref[0] = jnp.concatenate(
        [cost, jnp.full((_T, _QPAD - _Q), jnp.inf, jnp.float32)], axis=1)
    idxt_ref[0] = jax.lax.broadcasted_iota(jnp.int32, (_T, _K), 0)


def _tc_cost(pred_logits, pred_boxes, tgt_labels, tgt_boxes):
    pbT = pred_boxes.astype(jnp.float32).transpose(0, 2, 1)
    labels3 = tgt_labels.reshape(_B, _T, 1)
    return pl.pallas_call(
        _cost_kernel,
        grid=(_B,),
        in_specs=[
            pl.BlockSpec((1, _Q, _C), lambda b: (b, 0, 0)),
            pl.BlockSpec((1, 4, _Q), lambda b: (b, 0, 0)),
            pl.BlockSpec((1, _T, 1), lambda b: (b, 0, 0)),
            pl.BlockSpec((1, _T, 4), lambda b: (b, 0, 0)),
        ],
        out_specs=[
            pl.BlockSpec((1, _T, _QPAD), lambda b: (b, 0, 0)),
            pl.BlockSpec((1, _T, _K), lambda b: (b, 0, 0)),
        ],
        out_shape=[
            jax.ShapeDtypeStruct((_B, _T, _QPAD), jnp.float32),
            jax.ShapeDtypeStruct((_B, _T, _K), jnp.int32),
        ],
    )(pred_logits.astype(jnp.float32), pbT, labels3, tgt_boxes)


_GATHER_DNUMS = lax.GatherDimensionNumbers(
    offset_dims=(), collapsed_slice_dims=(0,), start_index_map=(0,))


def _permute(x, perm):
    return lax.gather(x, perm[:, None], _GATHER_DNUMS, (1,),
                      mode=lax.GatherScatterMode.PROMISE_IN_BOUNDS)


def _tree_min_idx(pairs):
    while len(pairs) > 1:
        nxt = []
        for a in range(0, len(pairs) - 1, 2):
            (va, ia), (vb, ib) = pairs[a], pairs[a + 1]
            pred = va <= vb
            nxt.append((jnp.where(pred, va, vb), jnp.where(pred, ia, ib)))
        if len(pairs) % 2:
            nxt.append(pairs[-1])
        pairs = nxt
    return pairs[0]


def _bfly_min(v, lane):
    for s in (1, 2, 4, 8):
        v = jnp.minimum(v, _permute(v, lane ^ s))
    return v


def _sc_topk_kernel(cost_hbm, out_hbm, buf, outbuf, sem0, sem1):
    lane = lax.broadcasted_iota(jnp.int32, (_L,), 0)
    wid = lax.axis_index("s") * 2 + lax.axis_index("c")
    for sl in range(2):
        for t in range(_BLK):
            for c in range(_NCHUNK, _NG * _GS):
                buf[sl, t, c] = jnp.full((_L,), jnp.inf, jnp.float32)

    def _start(blk, sl, sem):
        row0 = jnp.minimum(wid * _RPW + blk * _BLK, _ROWS - _BLK)
        pltpu.make_async_copy(
            cost_hbm.at[pl.ds(row0, _BLK)],
            buf.at[sl].at[:, pl.ds(0, _NCHUNK)], sem).start()

    def _wait(sl, sem):
        pltpu.make_async_copy(
            cost_hbm.at[pl.ds(0, _BLK)],
            buf.at[sl].at[:, pl.ds(0, _NCHUNK)], sem).wait()

    def _row_topk(sl, t, r):
        sbuf = buf.at[sl].at[t]
        gval = [None] * _NG
        gidx = [None] * _NG
        for k in range(_NG):
            gval[k], gidx[k] = _tree_min_idx(
                [(sbuf[k * _GS + u],
                  jnp.full((_L,), k * _GS + u, jnp.int32))
                 for u in range(_GS)])
        acc = jnp.zeros((_L,), jnp.int32)
        for j in range(_K):
            val, chk = _tree_min_idx(list(zip(gval, gidx)))
            gx = (chk << 4) + lane
            m = _bfly_min(val, lane)
            sel = jnp.where(val == m, gx, _NG * _GS * _L)
            ibest = _bfly_min(sel, lane)
            acc = jnp.where(lane == j, ibest, acc)
            s = ibest[0]
            c = s >> 4
            sbuf[c] = jnp.where(lane == (s & (_L - 1)), jnp.inf, sbuf[c])
            if j < _K - 1:
                kstar = s >> 7
                base = kstar << 3
                nval, nidx = _tree_min_idx(
                    [(sbuf[base + u],
                      jnp.full((_L,), base + u, jnp.int32))
                     for u in range(_GS)])
                for k in range(_NG):
                    keq = kstar == k
                    gval[k] = jnp.where(keq, nval, gval[k])
                    gidx[k] = jnp.where(keq, nidx, gidx[k])
        outbuf[r] = acc

    def _compute(sl, blk):
        for t in range(_BLK):
            _row_topk(sl, t, blk * _BLK + t)

    _start(0, 0, sem0)

    def pair_body(i, carry):
        b0 = 2 * i
        _wait(0, sem0)
        _start(b0 + 1, 1, sem1)
        _compute(0, b0)
        _wait(1, sem1)
        _start(b0 + 2, 0, sem0)
        _compute(1, b0 + 1)
        return carry

    lax.fori_loop(0, (_NBLK - 1) // 2, pair_body, 0)
    _wait(0, sem0)
    _compute(0, _NBLK - 1)
    pltpu.sync_copy(outbuf, out_hbm.at[wid])


def _sc_topk(cost):
    mesh = plsc.VectorSubcoreMesh(core_axis_name="c", subcore_axis_name="s")
    fn = functools.partial(
        pl.kernel, mesh=mesh,
        out_type=jax.ShapeDtypeStruct((_NW, _RPW, _L), jnp.int32),
        scratch_types=[
            pltpu.VMEM((2, _BLK, _NG * _GS, _L), jnp.float32),
            pltpu.VMEM((_RPW, _L), jnp.int32),
            pltpu.SemaphoreType.DMA,
            pltpu.SemaphoreType.DMA,
        ],
    )(_sc_topk_kernel)
    return fn(cost.reshape(_ROWS, _NCHUNK, _L))


def kernel(pred_logits, pred_boxes, tgt_labels, tgt_boxes):
    cost, out_t = _tc_cost(pred_logits, pred_boxes, tgt_labels, tgt_boxes)
    out_q = _sc_topk(cost)
    idx_q = out_q.reshape(_NW * _RPW, _L)[:_ROWS].reshape(
        _B, _T, _L)[:, :, :_K].transpose(0, 2, 1).reshape(_B, _K * _T)
    idx_t = out_t.reshape(_B, _K * _T)
    return idx_q, idx_t


# speedup vs baseline: 1.1248x; 1.1248x over previous
import functools

import jax
import jax.numpy as jnp
from jax import lax
from jax.experimental import pallas as pl
from jax.experimental.pallas import tpu as pltpu
from jax.experimental.pallas import tpu_sc as plsc

_B, _Q, _C, _T, _K = 8, 900, 91, 300, 6
_COST_CLASS, _COST_BBOX, _COST_GIOU = 1.0, 5.0, 2.0
_EPS = 1e-06
_ALPHA = 0.25
_L = 16
_NCHUNK = 57
_QPAD = _NCHUNK * _L
_NW = 32
_HB = 4
_HROWS = _HB * _T
_BLK = 4
_NBLK = 10
_RPW = _BLK * _NBLK
_NG = 8
_GS = 8


def _cost_kernel(logits_ref, pbT_ref, labels_ref, tb_ref, cost_ref, idxt_ref):
    logits = jnp.nan_to_num(logits_ref[0], nan=0.0)
    prob = jax.nn.sigmoid(logits)
    pos = _ALPHA * ((1.0 - prob) * (1.0 - prob)) * -jnp.log(prob + 1e-08)
    neg = (1.0 - _ALPHA) * (prob * prob) * -jnp.log(1.0 - prob + 1e-08)
    d = pos - neg
    labels = labels_ref[0]
    onehot = (labels == jax.lax.broadcasted_iota(jnp.int32, (_T, _C), 1)
              ).astype(jnp.float32)
    c_cls = jax.lax.dot_general(
        onehot, d, (((1,), (1,)), ((), ())),
        preferred_element_type=jnp.float32,
        precision=jax.lax.Precision.HIGHEST)

    pbT = jax.nn.sigmoid(pbT_ref[0])
    qcx, qcy = pbT[0:1, :], pbT[1:2, :]
    qw, qh = pbT[2:3, :], pbT[3:4, :]
    tb = jnp.clip(tb_ref[0], 0.0, 1.0)
    tx1, ty1 = tb[:, 0:1], tb[:, 1:2]
    tx2, ty2 = tb[:, 2:3], tb[:, 3:4]
    tw = jnp.maximum(tx2 - tx1, 1e-05)
    th = jnp.maximum(ty2 - ty1, 1e-05)
    tcx = (tx1 + tx2) * 0.5
    tcy = (ty1 + ty2) * 0.5

    c_l1 = (jnp.abs(qcx - tcx) + jnp.abs(qcy - tcy)
            + jnp.abs(qw - tw) + jnp.abs(qh - th))

    qx1 = jnp.clip(qcx - 0.5 * qw, 0.0, 1.0)
    qy1 = jnp.clip(qcy - 0.5 * qh, 0.0, 1.0)
    qx2 = jnp.clip(qcx + 0.5 * qw, 0.0, 1.0)
    qy2 = jnp.clip(qcy + 0.5 * qh, 0.0, 1.0)
    lt_x = jnp.maximum(qx1, tx1)
    lt_y = jnp.maximum(qy1, ty1)
    rb_x = jnp.minimum(qx2, tx2)
    rb_y = jnp.minimum(qy2, ty2)
    inter = jnp.maximum(rb_x - lt_x, 0.0) * jnp.maximum(rb_y - lt_y, 0.0)
    area_q = jnp.maximum(qx2 - qx1, 0.0) * jnp.maximum(qy2 - qy1, 0.0)
    area_t = jnp.maximum(tx2 - tx1, 0.0) * jnp.maximum(ty2 - ty1, 0.0)
    union = jnp.maximum(area_q + area_t - inter, _EPS)
    iou = inter / union
    en_x = jnp.maximum(qx2, tx2) - jnp.minimum(qx1, tx1)
    en_y = jnp.maximum(qy2, ty2) - jnp.minimum(qy1, ty1)
    area_c = jnp.maximum(jnp.maximum(en_x, 0.0) * jnp.maximum(en_y, 0.0), _EPS)
    giou = jnp.clip(iou - (area_c - union) / area_c, -1.0, 1.0)
    c_iou = 1.0 - giou

    cost = _COST_CLASS * c_cls + _COST_BBOX * c_l1 + _COST_GIOU * c_iou
    cost_ref[0] = jnp.concatenate(
        [cost, jnp.full((_T, _QPAD - _Q), jnp.inf, jnp.float32)], axis=1)
    idxt_ref[0] = jax.lax.broadcasted_iota(jnp.int32, (_T, _K), 0)


def _tc_cost(pred_logits, pred_boxes, tgt_labels, tgt_boxes, nb):
    pbT = pred_boxes.astype(jnp.float32).transpose(0, 2, 1)
    labels3 = tgt_labels.reshape(nb, _T, 1)
    return pl.pallas_call(
        _cost_kernel,
        grid=(nb,),
        in_specs=[
            pl.BlockSpec((1, _Q, _C), lambda b: (b, 0, 0)),
            pl.BlockSpec((1, 4, _Q), lambda b: (b, 0, 0)),
            pl.BlockSpec((1, _T, 1), lambda b: (b, 0, 0)),
            pl.BlockSpec((1, _T, 4), lambda b: (b, 0, 0)),
        ],
        out_specs=[
            pl.BlockSpec((1, _T, _QPAD), lambda b: (b, 0, 0)),
            pl.BlockSpec((1, _T, _K), lambda b: (b, 0, 0)),
        ],
        out_shape=[
            jax.ShapeDtypeStruct((nb, _T, _QPAD), jnp.float32),
            jax.ShapeDtypeStruct((nb, _T, _K), jnp.int32),
        ],
    )(pred_logits.astype(jnp.float32), pbT, labels3, tgt_boxes)


_GATHER_DNUMS = lax.GatherDimensionNumbers(
    offset_dims=(), collapsed_slice_dims=(0,), start_index_map=(0,))


def _permute(x, perm):
    return lax.gather(x, perm[:, None], _GATHER_DNUMS, (1,),
                      mode=lax.GatherScatterMode.PROMISE_IN_BOUNDS)


def _tree_min_idx(pairs):
    while len(pairs) > 1:
        nxt = []
        for a in range(0, len(pairs) - 1, 2):
            (va, ia), (vb, ib) = pairs[a], pairs[a + 1]
            pred = va <= vb
            nxt.append((jnp.where(pred, va, vb), jnp.where(pred, ia, ib)))
        if len(pairs) % 2:
            nxt.append(pairs[-1])
        pairs = nxt
    return pairs[0]


def _bfly_min(v, lane):
    for s in (1, 2, 4, 8):
        v = jnp.minimum(v, _permute(v, lane ^ s))
    return v


def _sc_topk_kernel(cost_hbm, out_hbm, buf, outbuf, sem0, sem1):
    lane = lax.broadcasted_iota(jnp.int32, (_L,), 0)
    wid = lax.axis_index("s") * 2 + lax.axis_index("c")

    def _start(blk, sl, sem):
        row0 = jnp.minimum(wid * _RPW + blk * _BLK, _HROWS - _BLK)
        pltpu.make_async_copy(
            cost_hbm.at[pl.ds(row0, _BLK)], buf.at[sl], sem).start()

    def _wait(sl, sem):
        pltpu.make_async_copy(
            cost_hbm.at[pl.ds(0, _BLK)], buf.at[sl], sem).wait()

    def _row_topk(sl, t, r):
        sbuf = buf.at[sl].at[t]
        gval = [None] * _NG
        gidx = [None] * _NG
        for k in range(_NG - 1):
            gval[k], gidx[k] = _tree_min_idx(
                [(sbuf[k * _GS + u],
                  jnp.full((_L,), k * _GS + u, jnp.int32))
                 for u in range(_GS)])
        gval[_NG - 1] = sbuf[_NCHUNK - 1]
        gidx[_NG - 1] = jnp.full((_L,), _NCHUNK - 1, jnp.int32)
        acc = jnp.zeros((_L,), jnp.int32)
        for j in range(_K):
            val, chk = _tree_min_idx(list(zip(gval, gidx)))
            gx = (chk << 4) + lane
            m = _bfly_min(val, lane)
            sel = jnp.where(val == m, gx, _NG * _GS * _L)
            ibest = _bfly_min(sel, lane)
            acc = jnp.where(lane == j, ibest, acc)
            s = ibest[0]
            c = s >> 4
            sbuf[c] = jnp.where(lane == (s & (_L - 1)), jnp.inf, sbuf[c])
            if j < _K - 1:
                kstar = s >> 7
                base = kstar << 3
                nval, nidx = _tree_min_idx(
                    [(sbuf[jnp.minimum(base + u, _NCHUNK - 1)],
                      jnp.full((_L,),
                               jnp.minimum(base + u, _NCHUNK - 1),
                               jnp.int32))
                     for u in range(_GS)])
                for k in range(_NG):
                    keq = kstar == k
                    gval[k] = jnp.where(keq, nval, gval[k])
                    gidx[k] = jnp.where(keq, nidx, gidx[k])
        outbuf[r] = acc

    def _compute(sl, blk):
        for t in range(_BLK):
            _row_topk(sl, t, blk * _BLK + t)

    _start(0, 0, sem0)

    def pair_body(i, carry):
        b0 = 2 * i
        _wait(0, sem0)
        _start(b0 + 1, 1, sem1)
        _compute(0, b0)
        _wait(1, sem1)
        _start(b0 + 2, 0, sem0)
        _compute(1, b0 + 1)
        return carry

    lax.fori_loop(0, _NBLK // 2, pair_body, 0)
    _wait(0, sem0)
    pltpu.sync_copy(outbuf, out_hbm.at[wid])


def _sc_topk(cost):
    mesh = plsc.VectorSubcoreMesh(core_axis_name="c", subcore_axis_name="s")
    fn = functools.partial(
        pl.kernel, mesh=mesh,
        out_type=jax.ShapeDtypeStruct((_NW, _RPW, _L), jnp.int32),
        scratch_types=[
            pltpu.VMEM((2, _BLK, _NCHUNK, _L), jnp.float32),
            pltpu.VMEM((_RPW, _L), jnp.int32),
            pltpu.SemaphoreType.DMA,
            pltpu.SemaphoreType.DMA,
        ],
    )(_sc_topk_kernel)
    return fn(cost.reshape(_HROWS, _NCHUNK, _L))


def kernel(pred_logits, pred_boxes, tgt_labels, tgt_boxes):
    outs_q, outs_t = [], []
    for h in range(_B // _HB):
        sl = slice(h * _HB, (h + 1) * _HB)
        cost, out_t = _tc_cost(pred_logits[sl], pred_boxes[sl],
                               tgt_labels[sl], tgt_boxes[sl], _HB)
        out_q = _sc_topk(cost)
        outs_q.append(out_q.reshape(_NW * _RPW, _L)[:_HROWS])
        outs_t.append(out_t)
    out_q = jnp.concatenate(outs_q, axis=0)
    idx_q = out_q.reshape(_B, _T, _L)[:, :, :_K].transpose(0, 2, 1).reshape(
        _B, _K * _T)
    idx_t = jnp.concatenate(outs_t, axis=0).reshape(_B, _K * _T)
    return idx_q, idx_t
